# baseline (speedup 1.0000x reference)
import math

import jax
import jax.numpy as jnp
from jax import lax
from jax.experimental import pallas as pl
from jax.experimental.pallas import tpu as pltpu
from jax.experimental.pallas import tpu_sc as plsc

D = 128
L = 16
SCALE = math.sqrt(float(D))
V = 1000
V_PAD = 1024
RB = 512
SC_ROWS = 12288


def _make_sc_kernel(B, S, NC, NS):
    NW = NC * NS
    b_per_w = S // NW
    CH = 128
    NCH = b_per_w // CH
    mesh = plsc.VectorSubcoreMesh(core_axis_name="c", subcore_axis_name="s")

    def body(x_hbm, step_hbm, pe_hbm, out_hbm, idx_v,
             xv0, xv1, pv0, pv1, ov0, ov1, ls0, ls1, ws0, ws1):
        wid = lax.axis_index("s") * NC + lax.axis_index("c")
        base = wid * b_per_w
        xv, pv, ov = (xv0, xv1), (pv0, pv1), (ov0, ov1)
        ls, ws = (ls0, ls1), (ws0, ws1)

        x0 = pltpu.async_copy(x_hbm.at[pl.ds(base, CH), :], xv[0], ls[0])
        x1 = pltpu.async_copy(x_hbm.at[pl.ds(base + CH, CH), :], xv[1], ls[1])
        pltpu.sync_copy(step_hbm.at[wid], idx_v)

        def issue(c):
            b = c & 1
            g = pltpu.async_copy(pe_hbm.at[idx_v.at[c]], pv[b], ls[b])
            xc = (pltpu.async_copy(x_hbm.at[pl.ds(base + c * CH, CH), :],
                                   xv[b], ls[b])
                  if c >= 2 else (x0 if c == 0 else x1))
            return (g, xc)

        pending = issue(0)
        wb = [None, None]
        for c in range(NCH):
            b = c & 1
            nxt = issue(c + 1) if c + 1 < NCH else None
            pending[0].wait()
            pending[1].wait()
            if wb[b] is not None:
                wb[b].wait()

            def row(r, carry, b=b):
                for g in range(D // L):
                    sl = pl.ds(g * L, L)
                    ov[b][r, sl] = xv[b][r, sl] * SCALE + pv[b][r, sl]
                return carry

            lax.fori_loop(0, CH, row, 0)
            wb[b] = pltpu.async_copy(ov[b],
                                     out_hbm.at[pl.ds(base + c * CH, CH), :],
                                     ws[b])
            pending = nxt
        wb[0].wait()
        wb[1].wait()

    buf = pltpu.VMEM((CH, D), jnp.float32)
    return pl.kernel(
        body,
        out_type=jax.ShapeDtypeStruct((B, D), jnp.float32),
        mesh=mesh,
        compiler_params=pltpu.CompilerParams(skip_device_barrier=True),
        scratch_types=[
            pltpu.VMEM((NCH, CH), jnp.int32),
            buf, buf, buf, buf, buf, buf,
            pltpu.SemaphoreType.DMA,
            pltpu.SemaphoreType.DMA,
            pltpu.SemaphoreType.DMA,
            pltpu.SemaphoreType.DMA,
        ],
    )


def _tc_body(x_ref, step_ref, pe_ref, out_ref):
    step = step_ref[...]
    onehot = (lax.broadcasted_iota(jnp.int32, (RB, V_PAD), 1)
              == step).astype(jnp.float32)
    gathered = jnp.dot(onehot, pe_ref[...], preferred_element_type=jnp.float32)
    out_ref[...] = x_ref[...] * SCALE + gathered


def _tc_kernel(x, step2, pe_pad, S):
    B = x.shape[0]
    nb = (B - S) // RB
    off = S // RB
    return pl.pallas_call(
        _tc_body,
        grid=(nb,),
        in_specs=[
            pl.BlockSpec((RB, D), lambda i: (i + off, 0)),
            pl.BlockSpec((RB, 1), lambda i: (i + off, 0)),
            pl.BlockSpec((V_PAD, D), lambda i: (0, 0)),
        ],
        out_specs=pl.BlockSpec((RB, D), lambda i: (i, 0)),
        out_shape=jax.ShapeDtypeStruct((B - S, D), jnp.float32),
    )(x, step2, pe_pad)


def kernel(x, step, pe):
    B = x.shape[0]
    S = SC_ROWS
    info = plsc.get_sparse_core_info()
    NC, NS = info.num_cores, info.num_subcores
    NW = NC * NS
    CH = 128
    step_i = step.astype(jnp.int32)

    step3 = step_i[:S].reshape(NW, (S // NW) // CH, CH)
    sc_out = _make_sc_kernel(B, S, NC, NS)(x, step3, pe)

    pe_pad = jnp.pad(pe, ((0, V_PAD - V), (0, 0)))
    tc_out = _tc_kernel(x, step_i.reshape(B, 1), pe_pad, S)

    return lax.dynamic_update_slice(sc_out, tc_out, (S, 0))


# speedup vs baseline: 1.1496x; 1.1496x over previous
import math

import jax
import jax.numpy as jnp
from jax import lax
from jax.experimental import pallas as pl
from jax.experimental.pallas import tpu as pltpu
from jax.experimental.pallas import tpu_sc as plsc

D = 128
L = 16
SCALE = math.sqrt(float(D))
V = 1000
V_PAD = 1024
RB = 512
SC_ROWS = 12288


def _make_sc_kernel(B, S, NC, NS):
    NW = NC * NS
    b_per_w = S // NW
    CH = 128
    NCH = b_per_w // CH
    mesh = plsc.VectorSubcoreMesh(core_axis_name="c", subcore_axis_name="s")

    def body(x_hbm, step_hbm, pe_hbm, out_hbm, idx_v,
             xv0, xv1, pv0, pv1, ov0, ov1, ls0, ls1, ws0, ws1):
        wid = lax.axis_index("s") * NC + lax.axis_index("c")
        base = wid * b_per_w
        xv, pv, ov = (xv0, xv1), (pv0, pv1), (ov0, ov1)
        ls, ws = (ls0, ls1), (ws0, ws1)

        x0 = pltpu.async_copy(x_hbm.at[pl.ds(base, CH), :], xv[0], ls[0])
        x1 = pltpu.async_copy(x_hbm.at[pl.ds(base + CH, CH), :], xv[1], ls[1])
        pltpu.sync_copy(step_hbm.at[wid], idx_v)

        def issue(c):
            b = c & 1
            g = pltpu.async_copy(pe_hbm.at[idx_v.at[c]], pv[b], ls[b])
            xc = (pltpu.async_copy(x_hbm.at[pl.ds(base + c * CH, CH), :],
                                   xv[b], ls[b])
                  if c >= 2 else (x0 if c == 0 else x1))
            return (g, xc)

        pending = issue(0)
        wb = [None, None]
        for c in range(NCH):
            b = c & 1
            nxt = issue(c + 1) if c + 1 < NCH else None
            pending[0].wait()
            pending[1].wait()
            if wb[b] is not None:
                wb[b].wait()

            def row(r, carry, b=b):
                for g in range(D // L):
                    sl = pl.ds(g * L, L)
                    ov[b][r, sl] = xv[b][r, sl] * SCALE + pv[b][r, sl]
                return carry

            lax.fori_loop(0, CH, row, 0)
            wb[b] = pltpu.async_copy(ov[b],
                                     out_hbm.at[pl.ds(base + c * CH, CH), :],
                                     ws[b])
            pending = nxt
        wb[0].wait()
        wb[1].wait()

    buf = pltpu.VMEM((CH, D), jnp.float32)
    return pl.kernel(
        body,
        out_type=jax.ShapeDtypeStruct((B, D), jnp.float32),
        mesh=mesh,
        compiler_params=pltpu.CompilerParams(skip_device_barrier=True),
        scratch_types=[
            pltpu.VMEM((NCH, CH), jnp.int32),
            buf, buf, buf, buf, buf, buf,
            pltpu.SemaphoreType.DMA,
            pltpu.SemaphoreType.DMA,
            pltpu.SemaphoreType.DMA,
            pltpu.SemaphoreType.DMA,
        ],
    )


def _tc_body(x_ref, step_ref, pe_ref, out_ref):
    step = step_ref[...]
    onehot_t = (lax.broadcasted_iota(jnp.int32, (V_PAD, RB), 0)
                == step).astype(jnp.float32)
    gathered = lax.dot_general(onehot_t, pe_ref[...],
                               dimension_numbers=(((0,), (0,)), ((), ())),
                               preferred_element_type=jnp.float32)
    out_ref[...] = x_ref[...] * SCALE + gathered


def _tc_kernel(x, step2, pe_pad, S):
    B = x.shape[0]
    nb = (B - S) // RB
    off = S // RB
    return pl.pallas_call(
        _tc_body,
        grid=(nb,),
        in_specs=[
            pl.BlockSpec((RB, D), lambda i: (i + off, 0)),
            pl.BlockSpec((1, RB), lambda i: (0, i + off)),
            pl.BlockSpec((V_PAD, D), lambda i: (0, 0)),
        ],
        out_specs=pl.BlockSpec((RB, D), lambda i: (i, 0)),
        out_shape=jax.ShapeDtypeStruct((B - S, D), jnp.float32),
    )(x, step2, pe_pad)


def kernel(x, step, pe):
    B = x.shape[0]
    S = SC_ROWS
    info = plsc.get_sparse_core_info()
    NC, NS = info.num_cores, info.num_subcores
    NW = NC * NS
    CH = 128
    step_i = step.astype(jnp.int32)

    step3 = step_i[:S].reshape(NW, (S // NW) // CH, CH)
    sc_out = _make_sc_kernel(B, S, NC, NS)(x, step3, pe)

    pe_pad = jnp.pad(pe, ((0, V_PAD - V), (0, 0)))
    tc_out = _tc_kernel(x, step_i.reshape(1, B), pe_pad, S)

    return lax.dynamic_update_slice(sc_out, tc_out, (S, 0))


# speedup vs baseline: 1.1972x; 1.0413x over previous
import math

import jax
import jax.numpy as jnp
from jax import lax
from jax.experimental import pallas as pl
from jax.experimental.pallas import tpu as pltpu
from jax.experimental.pallas import tpu_sc as plsc

D = 128
L = 16
SCALE = math.sqrt(float(D))


def _make_sc_kernel(B, NC, NS):
    NW = NC * NS
    b_per_w = B // NW
    CH = 128
    NCH = b_per_w // CH
    mesh = plsc.VectorSubcoreMesh(core_axis_name="c", subcore_axis_name="s")

    def body(x_hbm, step_hbm, pe_hbm, out_hbm, idx_v,
             xv0, xv1, pv0, pv1, ov0, ov1, ls0, ls1, ws0, ws1):
        wid = lax.axis_index("s") * NC + lax.axis_index("c")
        base = wid * b_per_w
        xv, pv, ov = (xv0, xv1), (pv0, pv1), (ov0, ov1)
        ls, ws = (ls0, ls1), (ws0, ws1)

        x0 = pltpu.async_copy(x_hbm.at[pl.ds(base, CH), :], xv[0], ls[0])
        x1 = pltpu.async_copy(x_hbm.at[pl.ds(base + CH, CH), :], xv[1], ls[1])
        pltpu.sync_copy(step_hbm.at[wid], idx_v)

        def issue(c):
            b = c & 1
            g = pltpu.async_copy(pe_hbm.at[idx_v.at[c]], pv[b], ls[b])
            xc = (pltpu.async_copy(x_hbm.at[pl.ds(base + c * CH, CH), :],
                                   xv[b], ls[b])
                  if c >= 2 else (x0 if c == 0 else x1))
            return (g, xc)

        pending = issue(0)
        wb = [None, None]
        for c in range(NCH):
            b = c & 1
            nxt = issue(c + 1) if c + 1 < NCH else None
            pending[0].wait()
            pending[1].wait()
            if wb[b] is not None:
                wb[b].wait()

            def row(r, carry, b=b):
                for g in range(D // L):
                    sl = pl.ds(g * L, L)
                    ov[b][r, sl] = xv[b][r, sl] * SCALE + pv[b][r, sl]
                return carry

            lax.fori_loop(0, CH, row, 0)
            wb[b] = pltpu.async_copy(ov[b],
                                     out_hbm.at[pl.ds(base + c * CH, CH), :],
                                     ws[b])
            pending = nxt
        wb[0].wait()
        wb[1].wait()

    buf = pltpu.VMEM((CH, D), jnp.float32)
    return pl.kernel(
        body,
        out_type=jax.ShapeDtypeStruct((B, D), jnp.float32),
        mesh=mesh,
        compiler_params=pltpu.CompilerParams(skip_device_barrier=True),
        scratch_types=[
            pltpu.VMEM((NCH, CH), jnp.int32),
            buf, buf, buf, buf, buf, buf,
            pltpu.SemaphoreType.DMA,
            pltpu.SemaphoreType.DMA,
            pltpu.SemaphoreType.DMA,
            pltpu.SemaphoreType.DMA,
        ],
    )


def kernel(x, step, pe):
    B = x.shape[0]
    info = plsc.get_sparse_core_info()
    NC, NS = info.num_cores, info.num_subcores
    NW = NC * NS
    b_per_w = B // NW
    CH = 128
    step3 = step.astype(jnp.int32).reshape(NW, b_per_w // CH, CH)
    return _make_sc_kernel(B, NC, NS)(x, step3, pe)
